# baseline (speedup 1.0000x reference)
import jax
import jax.numpy as jnp
from jax import lax
from jax.experimental import pallas as pl
from jax.experimental.pallas import tpu as pltpu
from jax.experimental.pallas import tpu_sc as plsc

B, S, V, M, D = 4096, 200, 1000000, 200, 64
N = B * S
NC, NS, L = 2, 16, 16
NW = NC * NS
ROWS_PER_W = N // NW
CHUNK = 256
NCHUNK = ROWS_PER_W // CHUNK
NSEG = CHUNK // 128
DV = D // L


def _body(seq_hbm, pos_hbm, tok_hbm, ptab_hbm, w_hbm, b_hbm, cf_hbm, out_hbm,
          idx0, idx1, pidx0, pidx1, tok0, tok1, pos0, pos1, ob0, ob1, posc, posc_sh,
          wv, bv, cfv,
          sgt0, sgt1, sgp0, sgp1, ss0, ss1, si0, si1):
    idxs, pidxs = [idx0, idx1], [pidx0, pidx1]
    toks, poss, obs = [tok0, tok1], [pos0, pos1], [ob0, ob1]
    sgt, sgp, ss, si = [sgt0, sgt1], [sgp0, sgp1], [ss0, ss1], [si0, si1]

    wid = lax.axis_index("s") * NC + lax.axis_index("c")
    base = wid * ROWS_PER_W

    pltpu.sync_copy(w_hbm, wv)
    pltpu.sync_copy(b_hbm, bv)
    pltpu.sync_copy(cf_hbm, cfv)
    w0 = [wv[pl.ds(j * L, L)] for j in range(DV)]
    w1 = [wv[pl.ds(D + j * L, L)] for j in range(DV)]
    bb = [bv[pl.ds(j * L, L)] for j in range(DV)]

    pltpu.sync_copy(ptab_hbm, posc)

    def posc_body(p, _):
        for j in range(DV):
            posc[p, pl.ds(j * L, L)] = posc[p, pl.ds(j * L, L)] * w1[j] + bb[j]
        return 0

    lax.fori_loop(0, M, posc_body, 0)

    @pl.when(lax.axis_index("s") == 0)
    def _():
        pltpu.sync_copy(posc, posc_sh)

    plsc.subcore_barrier()

    def fire_gathers(b):
        for k in range(NSEG):
            sl = pl.ds(k * 128, 128)
            pltpu.async_copy(tok_hbm.at[idxs[b].at[sl]], toks[b].at[sl], sgt[b])
            pltpu.async_copy(posc_sh.at[pidxs[b].at[sl]], poss[b].at[sl],
                             sgp[b])

    def wait_gathers(b):
        for k in range(NSEG):
            sl = pl.ds(k * 128, 128)
            pltpu.make_async_copy(tok_hbm.at[idxs[b].at[sl]], toks[b].at[sl],
                                  sgt[b]).wait()
            pltpu.make_async_copy(posc_sh.at[pidxs[b].at[sl]], poss[b].at[sl],
                                  sgp[b]).wait()

    def fire_idx(b, g):
        row0 = base + g * CHUNK
        pltpu.async_copy(seq_hbm.at[pl.ds(row0, CHUNK)], idxs[b], si[b])
        pltpu.async_copy(pos_hbm.at[pl.ds(row0, CHUNK)], pidxs[b], si[b])

    def wait_idx(b):
        pltpu.make_async_copy(seq_hbm.at[pl.ds(0, CHUNK)], idxs[b],
                              si[b]).wait()
        pltpu.make_async_copy(pos_hbm.at[pl.ds(0, CHUNK)], pidxs[b],
                              si[b]).wait()

    def fire_scatter(b, g):
        row0 = base + g * CHUNK
        s_idx = row0 // B
        bb0 = (row0 % B) // 128
        out0 = s_idx * (8 * 32 * 1024) + bb0 * 1024
        for k in range(8):
            pltpu.async_copy(obs[b].at[pl.ds(k * 2048, 2048)],
                             out_hbm.at[pl.ds(out0 + k * 32768, 2048)],
                             ss[b])

    def wait_scatter(b):
        for k in range(8):
            pltpu.make_async_copy(obs[b].at[pl.ds(k * 2048, 2048)],
                                  out_hbm.at[pl.ds(k * 2048, 2048)],
                                  ss[b]).wait()

    cflat = [cfv[pl.ds(j * L, L)] for j in range(DV)]

    def compute(b):
        def row_body(r, _):
            rowbase = (r // 128) * 1024 + (r % 128)
            for j in range(DV):
                t = toks[b][r, pl.ds(j * L, L)]
                p = poss[b][r, pl.ds(j * L, L)]
                plsc.store_scatter(obs[b], [cflat[j] + rowbase],
                                   t * w0[j] + p)
            return 0
        lax.fori_loop(0, CHUNK, row_body, 0)

    for b in range(2):
        row0 = base + b * CHUNK
        pltpu.sync_copy(seq_hbm.at[pl.ds(row0, CHUNK)], idxs[b])
        pltpu.sync_copy(pos_hbm.at[pl.ds(row0, CHUNK)], pidxs[b])
        fire_gathers(b)

    def pair_body(gp, _):
        for b in range(2):
            g = gp * 2 + b
            wait_gathers(b)
            pref = g + 2 < NCHUNK

            @pl.when(pref)
            def _():
                fire_idx(b, g + 2)

            @pl.when(g >= 2)
            def _():
                wait_scatter(b)

            compute(b)
            fire_scatter(b, g)

            @pl.when(pref)
            def _():
                wait_idx(b)
                fire_gathers(b)
        return 0

    lax.fori_loop(0, NCHUNK // 2, pair_body, 0)
    for b in range(2):
        wait_scatter(b)


@jax.jit
def _run(seq_flat, pos_flat, token_table, pos_table, w_flat, bias, cf):
    mesh = plsc.VectorSubcoreMesh(core_axis_name="c", subcore_axis_name="s")
    out = pl.kernel(
        _body,
        out_type=jax.ShapeDtypeStruct((S * 8 * 32 * 8 * 128,), jnp.float32),
        mesh=mesh,
        compiler_params=pltpu.CompilerParams(use_tc_tiling_on_sc=False,
                                             needs_layout_passes=False),
        scratch_types=[
            pltpu.VMEM((CHUNK,), jnp.int32),
            pltpu.VMEM((CHUNK,), jnp.int32),
            pltpu.VMEM((CHUNK,), jnp.int32),
            pltpu.VMEM((CHUNK,), jnp.int32),
            pltpu.VMEM((CHUNK, D), jnp.float32),
            pltpu.VMEM((CHUNK, D), jnp.float32),
            pltpu.VMEM((CHUNK, D), jnp.float32),
            pltpu.VMEM((CHUNK, D), jnp.float32),
            pltpu.VMEM((CHUNK * D,), jnp.float32),
            pltpu.VMEM((CHUNK * D,), jnp.float32),
            pltpu.VMEM((M, D), jnp.float32),
            pltpu.VMEM_SHARED((M, D), jnp.float32),
            pltpu.VMEM((2 * D,), jnp.float32),
            pltpu.VMEM((D,), jnp.float32),
            pltpu.VMEM((D,), jnp.int32),
            pltpu.SemaphoreType.DMA,
            pltpu.SemaphoreType.DMA,
            pltpu.SemaphoreType.DMA,
            pltpu.SemaphoreType.DMA,
            pltpu.SemaphoreType.DMA,
            pltpu.SemaphoreType.DMA,
            pltpu.SemaphoreType.DMA,
            pltpu.SemaphoreType.DMA,
        ],
    )(seq_flat, pos_flat, token_table, pos_table, w_flat, bias, cf)
    return out


def kernel(sequence, position_ids, token_table, pos_table, embedding_weights,
           embedding_bias):
    seq_flat = sequence.T.reshape(N).astype(jnp.int32)
    pos_flat = position_ids.T.reshape(N).astype(jnp.int32)
    w_flat = embedding_weights.reshape(2 * D).astype(jnp.float32)
    dd = jnp.arange(D, dtype=jnp.int32)
    cf = (dd // 8) * 2048 + (dd % 8) * 128
    out5 = _run(seq_flat, pos_flat, token_table, pos_table, w_flat,
                embedding_bias, cf).reshape(S, 8, 32, 8, 128)
    return out5.transpose(2, 4, 0, 1, 3).reshape(B, S, D)


# speedup vs baseline: 1.4796x; 1.4796x over previous
import jax
import jax.numpy as jnp
from jax import lax
from jax.experimental import pallas as pl
from jax.experimental.pallas import tpu as pltpu
from jax.experimental.pallas import tpu_sc as plsc

B, S, V, M, D = 4096, 200, 1000000, 200, 64
N = B * S
NC, NS, L = 2, 16, 16
NW = NC * NS
ROWS_PER_W = N // NW
CHUNK = 256
NCHUNK = ROWS_PER_W // CHUNK
NSEG = CHUNK // 128
DV = D // L


def _body(seq_hbm, pos_hbm, tok_hbm, ptab_hbm, w_hbm, b_hbm, out_hbm,
          idx0, idx1, pidx0, pidx1, tok0, tok1, pos0, pos1, ob0, ob1, posc, posc_sh,
          wv, bv,
          sgt0, sgt1, sgp0, sgp1, ss0, ss1, si0, si1):
    idxs, pidxs = [idx0, idx1], [pidx0, pidx1]
    toks, poss, obs = [tok0, tok1], [pos0, pos1], [ob0, ob1]
    sgt, sgp, ss, si = [sgt0, sgt1], [sgp0, sgp1], [ss0, ss1], [si0, si1]

    wid = lax.axis_index("s") * NC + lax.axis_index("c")
    base = wid * ROWS_PER_W

    pltpu.sync_copy(w_hbm, wv)
    pltpu.sync_copy(b_hbm, bv)
    w0 = [wv[pl.ds(j * L, L)] for j in range(DV)]
    w1 = [wv[pl.ds(D + j * L, L)] for j in range(DV)]
    bb = [bv[pl.ds(j * L, L)] for j in range(DV)]

    pltpu.sync_copy(ptab_hbm, posc)

    def posc_body(p, _):
        for j in range(DV):
            posc[p, pl.ds(j * L, L)] = posc[p, pl.ds(j * L, L)] * w1[j] + bb[j]
        return 0

    lax.fori_loop(0, M, posc_body, 0)

    @pl.when(lax.axis_index("s") == 0)
    def _():
        pltpu.sync_copy(posc, posc_sh)

    plsc.subcore_barrier()

    def fire_gathers(b):
        for k in range(NSEG):
            sl = pl.ds(k * 128, 128)
            pltpu.async_copy(tok_hbm.at[idxs[b].at[sl]], toks[b].at[sl], sgt[b])
            pltpu.async_copy(posc_sh.at[pidxs[b].at[sl]], poss[b].at[sl],
                             sgp[b])

    def wait_gathers(b):
        for k in range(NSEG):
            sl = pl.ds(k * 128, 128)
            pltpu.make_async_copy(tok_hbm.at[idxs[b].at[sl]], toks[b].at[sl],
                                  sgt[b]).wait()
            pltpu.make_async_copy(posc_sh.at[pidxs[b].at[sl]], poss[b].at[sl],
                                  sgp[b]).wait()

    def fire_idx(b, g):
        row0 = base + g * CHUNK
        pltpu.async_copy(seq_hbm.at[pl.ds(row0, CHUNK)], idxs[b], si[b])
        pltpu.async_copy(pos_hbm.at[pl.ds(row0, CHUNK)], pidxs[b], si[b])

    def wait_idx(b):
        pltpu.make_async_copy(seq_hbm.at[pl.ds(0, CHUNK)], idxs[b],
                              si[b]).wait()
        pltpu.make_async_copy(pos_hbm.at[pl.ds(0, CHUNK)], pidxs[b],
                              si[b]).wait()

    def fire_scatter(b, g):
        row0 = base + g * CHUNK
        pltpu.async_copy(obs[b], out_hbm.at[pl.ds(row0, CHUNK)], ss[b])

    def wait_scatter(b):
        pltpu.make_async_copy(obs[b], out_hbm.at[pl.ds(base, CHUNK)],
                              ss[b]).wait()

    def compute(b):
        def row_body(r, _):
            for j in range(DV):
                t = toks[b][r, pl.ds(j * L, L)]
                p = poss[b][r, pl.ds(j * L, L)]
                obs[b][r, pl.ds(j * L, L)] = t * w0[j] + p
            return 0
        lax.fori_loop(0, CHUNK, row_body, 0)

    for b in range(2):
        row0 = base + b * CHUNK
        pltpu.sync_copy(seq_hbm.at[pl.ds(row0, CHUNK)], idxs[b])
        pltpu.sync_copy(pos_hbm.at[pl.ds(row0, CHUNK)], pidxs[b])
        fire_gathers(b)

    def pair_body(gp, _):
        for b in range(2):
            g = gp * 2 + b
            wait_gathers(b)
            pref = g + 2 < NCHUNK

            @pl.when(pref)
            def _():
                fire_idx(b, g + 2)

            @pl.when(g >= 2)
            def _():
                wait_scatter(b)

            compute(b)
            fire_scatter(b, g)

            @pl.when(pref)
            def _():
                wait_idx(b)
                fire_gathers(b)
        return 0

    lax.fori_loop(0, NCHUNK // 2, pair_body, 0)
    for b in range(2):
        wait_scatter(b)


@jax.jit
def _run(seq_flat, pos_flat, token_table, pos_table, w_flat, bias):
    mesh = plsc.VectorSubcoreMesh(core_axis_name="c", subcore_axis_name="s")
    out = pl.kernel(
        _body,
        out_type=jax.ShapeDtypeStruct((N, D), jnp.float32),
        mesh=mesh,
        compiler_params=pltpu.CompilerParams(use_tc_tiling_on_sc=False,
                                             needs_layout_passes=False),
        scratch_types=[
            pltpu.VMEM((CHUNK,), jnp.int32),
            pltpu.VMEM((CHUNK,), jnp.int32),
            pltpu.VMEM((CHUNK,), jnp.int32),
            pltpu.VMEM((CHUNK,), jnp.int32),
            pltpu.VMEM((CHUNK, D), jnp.float32),
            pltpu.VMEM((CHUNK, D), jnp.float32),
            pltpu.VMEM((CHUNK, D), jnp.float32),
            pltpu.VMEM((CHUNK, D), jnp.float32),
            pltpu.VMEM((CHUNK, D), jnp.float32),
            pltpu.VMEM((CHUNK, D), jnp.float32),
            pltpu.VMEM((M, D), jnp.float32),
            pltpu.VMEM_SHARED((M, D), jnp.float32),
            pltpu.VMEM((2 * D,), jnp.float32),
            pltpu.VMEM((D,), jnp.float32),
            pltpu.SemaphoreType.DMA,
            pltpu.SemaphoreType.DMA,
            pltpu.SemaphoreType.DMA,
            pltpu.SemaphoreType.DMA,
            pltpu.SemaphoreType.DMA,
            pltpu.SemaphoreType.DMA,
            pltpu.SemaphoreType.DMA,
            pltpu.SemaphoreType.DMA,
        ],
    )(seq_flat, pos_flat, token_table, pos_table, w_flat, bias)
    return out


def kernel(sequence, position_ids, token_table, pos_table, embedding_weights,
           embedding_bias):
    seq_flat = sequence.reshape(N).astype(jnp.int32)
    pos_flat = position_ids.reshape(N).astype(jnp.int32)
    w_flat = embedding_weights.reshape(2 * D).astype(jnp.float32)
    out = _run(seq_flat, pos_flat, token_table, pos_table, w_flat,
               embedding_bias)
    return out.reshape(B, S, D)
